# baseline (speedup 1.0000x reference)
import functools

import jax
import jax.numpy as jnp
from jax import lax
from jax.experimental import pallas as pl
from jax.experimental.pallas import tpu as pltpu
from jax.experimental.pallas import tpu_sc as plsc

EPS = 1e-15

_N = 10000
_E = 320000
_NC = 2
_NS = 16
_CH = 64
_EPAD = 327680
_EALLOC = 339200
_FAST_CID = 1
_EPT_F = 16000
_EPT_S = 4480
_NCH_F = _EPT_F // _CH
_NCH_S = _EPT_S // _CH
_DEG_EPT = _EPAD // _NS
_DEG_NCH = 10240 // _CH
_PADN = 10240
_RPT = _PADN // _NS

_BLK = 1000


def _dot(a, b):
    return lax.dot_general(a, b, (((1,), (0,)), ((), ())),
                           preferred_element_type=jnp.float32)


def _tc_pre_body(x_ref, lvl_ref, emb_ref, ws_ref, wn_ref, b_ref,
                 self_ref, m_ref, cr0_ref):
    xb = x_ref[:]
    pid = pl.program_id(0)

    @pl.when(pid == 0)
    def _():
        ii = lax.broadcasted_iota(jnp.int32, (1, 128), 1)
        sgn = jnp.where(ii < 127, 1.0, -1.0).astype(jnp.float32)
        a = xb[0:1, :] * sgn
        bq = xb[1:2, :] * sgn
        i02 = jnp.sum(a * xb[2:3, :], keepdims=True)
        i13 = jnp.sum(bq * xb[3:4, :], keepdims=True)
        i03 = jnp.sum(a * xb[3:4, :], keepdims=True)
        i12 = jnp.sum(bq * xb[2:3, :], keepdims=True)
        cr0_ref[:, :] = (i02 * i13) / (i03 * i12 + EPS)

    nrm = jnp.sqrt(jnp.sum(xb * xb, axis=1, keepdims=True))
    xn = xb / (nrm + EPS)
    lv = lvl_ref[:]
    emb = jnp.where(lv == 0, emb_ref[0:1, :],
                    jnp.where(lv == 1, emb_ref[1:2, :], emb_ref[2:3, :]))
    self_ref[:] = (_dot(xn, ws_ref[0:128, :]) + _dot(emb, ws_ref[128:136, :])
                   + b_ref[:])
    m_ref[:] = _dot(xn, wn_ref[0:128, :]) + _dot(emb, wn_ref[128:136, :])


def _tc_pre(x, levels2d, lvl_emb, Ws, Wn, b):
    grid = _N // _BLK
    return pl.pallas_call(
        _tc_pre_body,
        grid=(grid,),
        in_specs=[
            pl.BlockSpec((_BLK, 128), lambda i: (i, 0)),
            pl.BlockSpec((_BLK, 1), lambda i: (i, 0)),
            pl.BlockSpec((3, 8), lambda i: (0, 0)),
            pl.BlockSpec((136, 128), lambda i: (0, 0)),
            pl.BlockSpec((136, 128), lambda i: (0, 0)),
            pl.BlockSpec((1, 128), lambda i: (0, 0)),
        ],
        out_specs=[
            pl.BlockSpec((_BLK, 128), lambda i: (i, 0)),
            pl.BlockSpec((_BLK, 128), lambda i: (i, 0)),
            pl.BlockSpec((1, 1), lambda i: (0, 0)),
        ],
        out_shape=[
            jax.ShapeDtypeStruct((_N, 128), jnp.float32),
            jax.ShapeDtypeStruct((_N, 128), jnp.float32),
            jax.ShapeDtypeStruct((1, 1), jnp.float32),
        ],
    )(x, levels2d, lvl_emb, Ws, Wn, b.reshape(1, 128))


def _tc_mid_body(self1_ref, p0_ref, p1_ref, deg0_ref, cr0_ref,
                 lvl_ref, emb_ref, ws_ref, wn_ref, b_ref,
                 self2_ref, m2_ref, s_ref):
    pid = pl.program_id(0)
    deg = jnp.maximum(deg0_ref[:, 0:1], 1.0)
    y = self1_ref[:] + (p0_ref[:] + p1_ref[:]) / deg
    nrm = jnp.sqrt(jnp.sum(y * y, axis=1, keepdims=True))
    y = y / (nrm + EPS)
    f = jnp.maximum(y, 0.0)
    fn = jnp.sqrt(jnp.sum(f * f, axis=1, keepdims=True))
    f = f / (fn + EPS)

    @pl.when(pid == 0)
    def _():
        i02 = jnp.sum(f[0:1, :] * f[2:3, :], keepdims=True) - 1.0
        i13 = jnp.sum(f[1:2, :] * f[3:4, :], keepdims=True) - 1.0
        i03 = jnp.sum(f[0:1, :] * f[3:4, :], keepdims=True) - 1.0
        i12 = jnp.sum(f[1:2, :] * f[2:3, :], keepdims=True) - 1.0
        cr1 = (i02 * i13) / (i03 * i12 + EPS)
        cr0 = cr0_ref[:, :]
        valid = jnp.logical_and(jnp.logical_and(cr1 == cr1, cr0 == cr0),
                                cr1 != 0.0)
        safe = jnp.where(valid, cr1, 1.0)
        s = jnp.where(valid, jnp.sqrt(jnp.abs(cr0 / safe) + EPS), 1.0)
        s_ref[0] = s[0, 0]

    g = f * s_ref[0]
    lv = lvl_ref[:]
    emb = jnp.where(lv == 0, emb_ref[0:1, :],
                    jnp.where(lv == 1, emb_ref[1:2, :], emb_ref[2:3, :]))
    self2_ref[:] = (_dot(g, ws_ref[0:128, :]) + _dot(emb, ws_ref[128:136, :])
                    + b_ref[:])
    m2_ref[:] = _dot(g, wn_ref[0:128, :]) + _dot(emb, wn_ref[128:136, :])


def _tc_mid(self1, p0, p1, deg0, cr0, levels2d, lvl_emb, Ws, Wn, b):
    grid = _N // _BLK
    return pl.pallas_call(
        _tc_mid_body,
        grid=(grid,),
        in_specs=[
            pl.BlockSpec((_BLK, 128), lambda i: (i, 0)),
            pl.BlockSpec((_BLK, 128), lambda i: (i, 0)),
            pl.BlockSpec((_BLK, 128), lambda i: (i, 0)),
            pl.BlockSpec((_BLK, 128), lambda i: (i, 0)),
            pl.BlockSpec((1, 1), lambda i: (0, 0)),
            pl.BlockSpec((_BLK, 1), lambda i: (i, 0)),
            pl.BlockSpec((3, 8), lambda i: (0, 0)),
            pl.BlockSpec((136, 128), lambda i: (0, 0)),
            pl.BlockSpec((136, 128), lambda i: (0, 0)),
            pl.BlockSpec((1, 128), lambda i: (0, 0)),
        ],
        out_specs=[
            pl.BlockSpec((_BLK, 128), lambda i: (i, 0)),
            pl.BlockSpec((_BLK, 128), lambda i: (i, 0)),
        ],
        out_shape=[
            jax.ShapeDtypeStruct((_N, 128), jnp.float32),
            jax.ShapeDtypeStruct((_N, 128), jnp.float32),
        ],
        scratch_shapes=[pltpu.SMEM((1,), jnp.float32)],
    )(self1, p0, p1, deg0, cr0, levels2d, lvl_emb, Ws, Wn,
      b.reshape(1, 128))


def _tc_post_body(self2_ref, q0_ref, q1_ref, deg0_ref, out_ref):
    deg = jnp.maximum(deg0_ref[:, 0:1], 1.0)
    z = self2_ref[:] + (q0_ref[:] + q1_ref[:]) / deg
    nrm = jnp.sqrt(jnp.sum(z * z, axis=1, keepdims=True))
    z = z / (nrm + EPS)
    nrm2 = jnp.sqrt(jnp.sum(z * z, axis=1, keepdims=True))
    out_ref[:] = z / (nrm2 + EPS)


def _tc_post(self2, q0, q1, deg0):
    grid = _N // _BLK
    return pl.pallas_call(
        _tc_post_body,
        grid=(grid,),
        in_specs=[
            pl.BlockSpec((_BLK, 128), lambda i: (i, 0)),
            pl.BlockSpec((_BLK, 128), lambda i: (i, 0)),
            pl.BlockSpec((_BLK, 128), lambda i: (i, 0)),
            pl.BlockSpec((_BLK, 128), lambda i: (i, 0)),
        ],
        out_specs=pl.BlockSpec((_BLK, 128), lambda i: (i, 0)),
        out_shape=jax.ShapeDtypeStruct((_N, 128), jnp.float32),
    )(self2, q0, q1, deg0)


def _make_sc_agg(with_deg):
    out_type = [jax.ShapeDtypeStruct((_NC * _PADN, 128), jnp.float32)]
    scratch = [
        pltpu.VMEM((_EPT_F,), jnp.int32),
        pltpu.VMEM((_EPT_F,), jnp.int32),
        pltpu.VMEM((_CH, 128), jnp.float32),
        pltpu.VMEM((_CH, 128), jnp.float32),
        pltpu.VMEM_SHARED((_PADN, 128), jnp.float32),
        pltpu.SemaphoreType.DMA,
        pltpu.SemaphoreType.DMA,
    ]
    if with_deg:
        out_type.append(jax.ShapeDtypeStruct((_PADN, 128), jnp.float32))
    mesh = plsc.VectorSubcoreMesh(core_axis_name="c", subcore_axis_name="s",
                                  num_cores=_NC, num_subcores=_NS)

    def body(m_hbm, src_hbm, dst_hbm, zeros_hbm, ones_hbm, *rest):
        if with_deg:
            (acc_out, deg_out, s_all, d_all, rows0, rows1, acc_sh,
             sem0, sem1) = rest
        else:
            acc_out, s_all, d_all, rows0, rows1, acc_sh, sem0, sem1 = rest
        cid = lax.axis_index("c")
        tid = lax.axis_index("s")
        is_fast = cid == _FAST_CID
        ept = jnp.where(is_fast, _EPT_F, _EPT_S)
        nch = jnp.where(is_fast, _NCH_F, _NCH_S)
        cbase = jnp.where(is_fast, tid * _EPT_F,
                          _NS * _EPT_F + tid * _EPT_S)
        for k in range(_RPT // _CH):
            pltpu.sync_copy(zeros_hbm,
                            acc_sh.at[pl.ds(tid * _RPT + k * _CH, _CH)])
        pltpu.sync_copy(src_hbm.at[pl.ds(cbase, _EPT_F)],
                        s_all)
        pltpu.sync_copy(dst_hbm.at[pl.ds(cbase, _EPT_F)],
                        d_all)
        plsc.subcore_barrier()

        pltpu.async_copy(m_hbm.at[s_all.at[pl.ds(0, _CH)]], rows0, sem0)

        def step(j, c):
            c0 = 2 * j * _CH
            s0 = s_all.at[pl.ds(c0, _CH)]
            s1 = s_all.at[pl.ds(c0 + _CH, _CH)]
            pltpu.async_copy(m_hbm.at[s1], rows1, sem1)
            pltpu.make_async_copy(m_hbm.at[s0], rows0, sem0).wait()
            pltpu.sync_copy(rows0, acc_sh.at[d_all.at[pl.ds(c0, _CH)]],
                            add=True)

            @pl.when(2 * j + 2 < nch)
            def _():
                pltpu.async_copy(
                    m_hbm.at[s_all.at[pl.ds(c0 + 2 * _CH, _CH)]], rows0, sem0)

            pltpu.make_async_copy(m_hbm.at[s1], rows1, sem1).wait()
            pltpu.sync_copy(rows1, acc_sh.at[d_all.at[pl.ds(c0 + _CH, _CH)]],
                            add=True)
            return c

        lax.fori_loop(0, nch // 2, step, 0)
        plsc.subcore_barrier()
        pltpu.sync_copy(acc_sh.at[pl.ds(tid * _RPT, _RPT)],
                        acc_out.at[pl.ds(cid * _PADN + tid * _RPT, _RPT)])

        if with_deg:
            @pl.when(jnp.logical_not(is_fast))
            def _():
                for k in range(_RPT // _CH):
                    pltpu.sync_copy(zeros_hbm,
                                    acc_sh.at[pl.ds(tid * _RPT + k * _CH,
                                                    _CH)])
                pltpu.sync_copy(ones_hbm, rows0)
                plsc.subcore_barrier()
                for half in range(2):
                    pltpu.sync_copy(
                        dst_hbm.at[pl.ds(tid * _DEG_EPT + half * 10240,
                                         10240)],
                        d_all.at[pl.ds(0, 10240)])

                    def step2(j, c):
                        for k in range(8):
                            pltpu.async_copy(
                                rows0,
                                acc_sh.at[d_all.at[pl.ds((8 * j + k) * _CH,
                                                         _CH)]],
                                sem0, add=True)
                        for k in range(8):
                            pltpu.make_async_copy(
                                rows0,
                                acc_sh.at[d_all.at[pl.ds((8 * j + k) * _CH,
                                                         _CH)]],
                                sem0).wait()
                        return c

                    lax.fori_loop(0, _DEG_NCH // 8, step2, 0)
                plsc.subcore_barrier()
                pltpu.sync_copy(acc_sh.at[pl.ds(tid * _RPT, _RPT)],
                                deg_out.at[pl.ds(tid * _RPT, _RPT)])

    return pl.kernel(body, out_type=tuple(out_type) if with_deg else out_type[0],
                     mesh=mesh, scratch_types=scratch)


_sc_agg_deg = _make_sc_agg(True)
_sc_agg = _make_sc_agg(False)


def kernel(x, edge_index, node_levels, W_self1, W_neigh1, b1, lvl_emb1,
           W_self2, W_neigh2, b2, lvl_emb2):
    src = edge_index[0]
    dst = edge_index[1]
    pad = _EALLOC - _E
    srcp = jnp.concatenate([src, jnp.zeros((pad,), jnp.int32)])
    dstp = jnp.concatenate([dst, jnp.full((pad,), _N, jnp.int32)])
    levels2d = node_levels.reshape(_N, 1)
    zeros_hbm = jnp.zeros((_CH, 128), jnp.float32)

    self1, m1, cr0 = _tc_pre(x, levels2d, lvl_emb1, W_self1, W_neigh1, b1)
    ones_hbm = jnp.ones((_CH, 128), jnp.float32)
    acc1, degp = _sc_agg_deg(m1, srcp, dstp, zeros_hbm, ones_hbm)
    acc1 = acc1.reshape(_NC, _PADN, 128)
    deg0 = degp[:_N]

    self2, m2 = _tc_mid(self1, acc1[0, :_N], acc1[1, :_N], deg0, cr0,
                        levels2d, lvl_emb2, W_self2, W_neigh2, b2)
    acc2 = _sc_agg(m2, srcp, dstp, zeros_hbm, ones_hbm).reshape(_NC, _PADN, 128)
    return _tc_post(self2, acc2[0, :_N], acc2[1, :_N], deg0)


# speedup vs baseline: 1.1678x; 1.1678x over previous
import functools

import jax
import jax.numpy as jnp
from jax import lax
from jax.experimental import pallas as pl
from jax.experimental.pallas import tpu as pltpu
from jax.experimental.pallas import tpu_sc as plsc

EPS = 1e-15

_N = 10000
_E = 320000
_NC = 2
_NS = 16
_CH = 64
_EPAD = 327680
_EALLOC = 339200
_FAST_CID = 1
_EPT_F = 16000
_EPT_S = 4480
_NCH_F = _EPT_F // _CH
_NCH_S = _EPT_S // _CH
_DEG_EPT = _EPAD // (_NC * _NS)
_DEG_NCH = _DEG_EPT // _CH
_PADN = 10240
_RPT = _PADN // _NS

_BLK = 1000


def _dot(a, b):
    return lax.dot_general(a, b, (((1,), (0,)), ((), ())),
                           preferred_element_type=jnp.float32)


def _tc_pre_body(x_ref, lvl_ref, emb_ref, ws_ref, wn_ref, b_ref,
                 self_ref, m_ref, cr0_ref):
    xb = x_ref[:]
    pid = pl.program_id(0)

    @pl.when(pid == 0)
    def _():
        ii = lax.broadcasted_iota(jnp.int32, (1, 128), 1)
        sgn = jnp.where(ii < 127, 1.0, -1.0).astype(jnp.float32)
        a = xb[0:1, :] * sgn
        bq = xb[1:2, :] * sgn
        i02 = jnp.sum(a * xb[2:3, :], keepdims=True)
        i13 = jnp.sum(bq * xb[3:4, :], keepdims=True)
        i03 = jnp.sum(a * xb[3:4, :], keepdims=True)
        i12 = jnp.sum(bq * xb[2:3, :], keepdims=True)
        cr0_ref[:, :] = (i02 * i13) / (i03 * i12 + EPS)

    nrm = jnp.sqrt(jnp.sum(xb * xb, axis=1, keepdims=True))
    xn = xb / (nrm + EPS)
    lv = lvl_ref[:]
    emb = jnp.where(lv == 0, emb_ref[0:1, :],
                    jnp.where(lv == 1, emb_ref[1:2, :], emb_ref[2:3, :]))
    self_ref[:] = (_dot(xn, ws_ref[0:128, :]) + _dot(emb, ws_ref[128:136, :])
                   + b_ref[:])
    m_ref[:] = _dot(xn, wn_ref[0:128, :]) + _dot(emb, wn_ref[128:136, :])


def _tc_pre(x, levels2d, lvl_emb, Ws, Wn, b):
    grid = _N // _BLK
    return pl.pallas_call(
        _tc_pre_body,
        grid=(grid,),
        in_specs=[
            pl.BlockSpec((_BLK, 128), lambda i: (i, 0)),
            pl.BlockSpec((_BLK, 1), lambda i: (i, 0)),
            pl.BlockSpec((3, 8), lambda i: (0, 0)),
            pl.BlockSpec((136, 128), lambda i: (0, 0)),
            pl.BlockSpec((136, 128), lambda i: (0, 0)),
            pl.BlockSpec((1, 128), lambda i: (0, 0)),
        ],
        out_specs=[
            pl.BlockSpec((_BLK, 128), lambda i: (i, 0)),
            pl.BlockSpec((_BLK, 128), lambda i: (i, 0)),
            pl.BlockSpec((1, 1), lambda i: (0, 0)),
        ],
        out_shape=[
            jax.ShapeDtypeStruct((_N, 128), jnp.float32),
            jax.ShapeDtypeStruct((_N, 128), jnp.float32),
            jax.ShapeDtypeStruct((1, 1), jnp.float32),
        ],
    )(x, levels2d, lvl_emb, Ws, Wn, b.reshape(1, 128))


def _tc_mid_body(self1_ref, p0_ref, p1_ref, deg0_ref, deg1_ref, cr0_ref,
                 lvl_ref, emb_ref, ws_ref, wn_ref, b_ref,
                 self2_ref, m2_ref, s_ref):
    pid = pl.program_id(0)
    deg = jnp.maximum(deg0_ref[:, 0:1] + deg1_ref[:, 0:1], 1.0)
    y = self1_ref[:] + (p0_ref[:] + p1_ref[:]) / deg
    nrm = jnp.sqrt(jnp.sum(y * y, axis=1, keepdims=True))
    y = y / (nrm + EPS)
    f = jnp.maximum(y, 0.0)
    fn = jnp.sqrt(jnp.sum(f * f, axis=1, keepdims=True))
    f = f / (fn + EPS)

    @pl.when(pid == 0)
    def _():
        i02 = jnp.sum(f[0:1, :] * f[2:3, :], keepdims=True) - 1.0
        i13 = jnp.sum(f[1:2, :] * f[3:4, :], keepdims=True) - 1.0
        i03 = jnp.sum(f[0:1, :] * f[3:4, :], keepdims=True) - 1.0
        i12 = jnp.sum(f[1:2, :] * f[2:3, :], keepdims=True) - 1.0
        cr1 = (i02 * i13) / (i03 * i12 + EPS)
        cr0 = cr0_ref[:, :]
        valid = jnp.logical_and(jnp.logical_and(cr1 == cr1, cr0 == cr0),
                                cr1 != 0.0)
        safe = jnp.where(valid, cr1, 1.0)
        s = jnp.where(valid, jnp.sqrt(jnp.abs(cr0 / safe) + EPS), 1.0)
        s_ref[0] = s[0, 0]

    g = f * s_ref[0]
    lv = lvl_ref[:]
    emb = jnp.where(lv == 0, emb_ref[0:1, :],
                    jnp.where(lv == 1, emb_ref[1:2, :], emb_ref[2:3, :]))
    self2_ref[:] = (_dot(g, ws_ref[0:128, :]) + _dot(emb, ws_ref[128:136, :])
                    + b_ref[:])
    m2_ref[:] = _dot(g, wn_ref[0:128, :]) + _dot(emb, wn_ref[128:136, :])


def _tc_mid(self1, p0, p1, deg0, deg1, cr0, levels2d, lvl_emb, Ws, Wn, b):
    grid = _N // _BLK
    return pl.pallas_call(
        _tc_mid_body,
        grid=(grid,),
        in_specs=[
            pl.BlockSpec((_BLK, 128), lambda i: (i, 0)),
            pl.BlockSpec((_BLK, 128), lambda i: (i, 0)),
            pl.BlockSpec((_BLK, 128), lambda i: (i, 0)),
            pl.BlockSpec((_BLK, 128), lambda i: (i, 0)),
            pl.BlockSpec((_BLK, 128), lambda i: (i, 0)),
            pl.BlockSpec((1, 1), lambda i: (0, 0)),
            pl.BlockSpec((_BLK, 1), lambda i: (i, 0)),
            pl.BlockSpec((3, 8), lambda i: (0, 0)),
            pl.BlockSpec((136, 128), lambda i: (0, 0)),
            pl.BlockSpec((136, 128), lambda i: (0, 0)),
            pl.BlockSpec((1, 128), lambda i: (0, 0)),
        ],
        out_specs=[
            pl.BlockSpec((_BLK, 128), lambda i: (i, 0)),
            pl.BlockSpec((_BLK, 128), lambda i: (i, 0)),
        ],
        out_shape=[
            jax.ShapeDtypeStruct((_N, 128), jnp.float32),
            jax.ShapeDtypeStruct((_N, 128), jnp.float32),
        ],
        scratch_shapes=[pltpu.SMEM((1,), jnp.float32)],
    )(self1, p0, p1, deg0, deg1, cr0, levels2d, lvl_emb, Ws, Wn,
      b.reshape(1, 128))


def _tc_post_body(self2_ref, q0_ref, q1_ref, deg0_ref, deg1_ref, out_ref):
    deg = jnp.maximum(deg0_ref[:, 0:1] + deg1_ref[:, 0:1], 1.0)
    z = self2_ref[:] + (q0_ref[:] + q1_ref[:]) / deg
    nrm = jnp.sqrt(jnp.sum(z * z, axis=1, keepdims=True))
    z = z / (nrm + EPS)
    nrm2 = jnp.sqrt(jnp.sum(z * z, axis=1, keepdims=True))
    out_ref[:] = z / (nrm2 + EPS)


def _tc_post(self2, q0, q1, deg0, deg1):
    grid = _N // _BLK
    return pl.pallas_call(
        _tc_post_body,
        grid=(grid,),
        in_specs=[
            pl.BlockSpec((_BLK, 128), lambda i: (i, 0)),
            pl.BlockSpec((_BLK, 128), lambda i: (i, 0)),
            pl.BlockSpec((_BLK, 128), lambda i: (i, 0)),
            pl.BlockSpec((_BLK, 128), lambda i: (i, 0)),
            pl.BlockSpec((_BLK, 128), lambda i: (i, 0)),
        ],
        out_specs=pl.BlockSpec((_BLK, 128), lambda i: (i, 0)),
        out_shape=jax.ShapeDtypeStruct((_N, 128), jnp.float32),
    )(self2, q0, q1, deg0, deg1)


def _make_sc_agg(with_deg):
    out_type = [jax.ShapeDtypeStruct((_NC * _PADN, 128), jnp.float32)]
    scratch = [
        pltpu.VMEM((_EPT_F,), jnp.int32),
        pltpu.VMEM((_EPT_F,), jnp.int32),
        pltpu.VMEM((_CH, 128), jnp.float32),
        pltpu.VMEM((_CH, 128), jnp.float32),
        pltpu.VMEM_SHARED((_PADN, 128), jnp.float32),
        pltpu.SemaphoreType.DMA,
        pltpu.SemaphoreType.DMA,
    ]
    if with_deg:
        out_type.append(jax.ShapeDtypeStruct((_NC * _PADN, 128), jnp.float32))
    mesh = plsc.VectorSubcoreMesh(core_axis_name="c", subcore_axis_name="s",
                                  num_cores=_NC, num_subcores=_NS)

    def body(m_hbm, src_hbm, dst_hbm, zeros_hbm, ones_hbm, *rest):
        if with_deg:
            (acc_out, deg_out, s_all, d_all, rows0, rows1, acc_sh,
             sem0, sem1) = rest
        else:
            acc_out, s_all, d_all, rows0, rows1, acc_sh, sem0, sem1 = rest
        cid = lax.axis_index("c")
        tid = lax.axis_index("s")
        is_fast = cid == _FAST_CID
        ept = jnp.where(is_fast, _EPT_F, _EPT_S)
        nch = jnp.where(is_fast, _NCH_F, _NCH_S)
        cbase = jnp.where(is_fast, tid * _EPT_F,
                          _NS * _EPT_F + tid * _EPT_S)
        for k in range(_RPT // _CH):
            pltpu.sync_copy(zeros_hbm,
                            acc_sh.at[pl.ds(tid * _RPT + k * _CH, _CH)])
        pltpu.sync_copy(src_hbm.at[pl.ds(cbase, _EPT_F)],
                        s_all)
        pltpu.sync_copy(dst_hbm.at[pl.ds(cbase, _EPT_F)],
                        d_all)
        plsc.subcore_barrier()

        pltpu.async_copy(m_hbm.at[s_all.at[pl.ds(0, _CH)]], rows0, sem0)

        def step(j, c):
            c0 = 2 * j * _CH
            s0 = s_all.at[pl.ds(c0, _CH)]
            s1 = s_all.at[pl.ds(c0 + _CH, _CH)]
            pltpu.async_copy(m_hbm.at[s1], rows1, sem1)
            pltpu.make_async_copy(m_hbm.at[s0], rows0, sem0).wait()
            pltpu.sync_copy(rows0, acc_sh.at[d_all.at[pl.ds(c0, _CH)]],
                            add=True)

            @pl.when(2 * j + 2 < nch)
            def _():
                pltpu.async_copy(
                    m_hbm.at[s_all.at[pl.ds(c0 + 2 * _CH, _CH)]], rows0, sem0)

            pltpu.make_async_copy(m_hbm.at[s1], rows1, sem1).wait()
            pltpu.sync_copy(rows1, acc_sh.at[d_all.at[pl.ds(c0 + _CH, _CH)]],
                            add=True)
            return c

        lax.fori_loop(0, nch // 2, step, 0)
        plsc.subcore_barrier()
        pltpu.sync_copy(acc_sh.at[pl.ds(tid * _RPT, _RPT)],
                        acc_out.at[pl.ds(cid * _PADN + tid * _RPT, _RPT)])

        if with_deg:
            wid = cid * _NS + tid
            for k in range(_RPT // _CH):
                pltpu.sync_copy(zeros_hbm,
                                acc_sh.at[pl.ds(tid * _RPT + k * _CH, _CH)])
            pltpu.sync_copy(dst_hbm.at[pl.ds(wid * _DEG_EPT, _DEG_EPT)],
                            d_all.at[pl.ds(0, _DEG_EPT)])
            pltpu.sync_copy(ones_hbm, rows0)
            plsc.subcore_barrier()

            def step2(j, c):
                for k in range(8):
                    pltpu.async_copy(
                        rows0,
                        acc_sh.at[d_all.at[pl.ds((8 * j + k) * _CH, _CH)]],
                        sem0, add=True)
                for k in range(8):
                    pltpu.make_async_copy(
                        rows0,
                        acc_sh.at[d_all.at[pl.ds((8 * j + k) * _CH, _CH)]],
                        sem0).wait()
                return c

            lax.fori_loop(0, _DEG_NCH // 8, step2, 0)
            plsc.subcore_barrier()
            pltpu.sync_copy(acc_sh.at[pl.ds(tid * _RPT, _RPT)],
                            deg_out.at[pl.ds(cid * _PADN + tid * _RPT, _RPT)])

    return pl.kernel(body, out_type=tuple(out_type) if with_deg else out_type[0],
                     mesh=mesh, scratch_types=scratch)


_sc_agg_deg = _make_sc_agg(True)
_sc_agg = _make_sc_agg(False)


def kernel(x, edge_index, node_levels, W_self1, W_neigh1, b1, lvl_emb1,
           W_self2, W_neigh2, b2, lvl_emb2):
    src = edge_index[0]
    dst = edge_index[1]
    pad = _EALLOC - _E
    srcp = jnp.concatenate([src, jnp.zeros((pad,), jnp.int32)])
    dstp = jnp.concatenate([dst, jnp.full((pad,), _N, jnp.int32)])
    levels2d = node_levels.reshape(_N, 1)
    zeros_hbm = jnp.zeros((_CH, 128), jnp.float32)

    self1, m1, cr0 = _tc_pre(x, levels2d, lvl_emb1, W_self1, W_neigh1, b1)
    ones_hbm = jnp.ones((_CH, 128), jnp.float32)
    acc1, degp = _sc_agg_deg(m1, srcp, dstp, zeros_hbm, ones_hbm)
    acc1 = acc1.reshape(_NC, _PADN, 128)
    degp = degp.reshape(_NC, _PADN, 128)
    deg0 = degp[0, :_N]
    deg1 = degp[1, :_N]

    self2, m2 = _tc_mid(self1, acc1[0, :_N], acc1[1, :_N], deg0, deg1, cr0,
                        levels2d, lvl_emb2, W_self2, W_neigh2, b2)
    acc2 = _sc_agg(m2, srcp, dstp, zeros_hbm, ones_hbm).reshape(_NC, _PADN, 128)
    return _tc_post(self2, acc2[0, :_N], acc2[1, :_N], deg0, deg1)
